# Initial kernel scaffold; baseline (speedup 1.0000x reference)
import jax
import jax.numpy as jnp
from jax.experimental import pallas as pl


def kernel(x, edge_index, W, b, gamma, beta):
    raise NotImplementedError("write your pallas kernel here")


# speedup vs baseline: 10.5589x; 10.5589x over previous
import functools

import jax
import jax.numpy as jnp
from jax import lax
from jax.experimental import pallas as pl
from jax.experimental.pallas import tpu as pltpu
from jax.experimental.pallas import tpu_sc as plsc

N = 1000
NP = 1024
E = 16000
NW = 32
PER = 512
EP = NW * PER
TILE_WORDS = (NP * NP) // 16
ZCHUNK = 8192


def _sc_build_a0(src_p, dst_p, w_p):
    mesh = plsc.VectorSubcoreMesh(core_axis_name="c", subcore_axis_name="s")

    @functools.partial(
        pl.kernel,
        mesh=mesh,
        out_type=jax.ShapeDtypeStruct((2, 16, TILE_WORDS), jnp.float32),
        scratch_types=[
            pltpu.VMEM((PER,), jnp.int32),
            pltpu.VMEM((PER,), jnp.int32),
            pltpu.VMEM((PER,), jnp.float32),
            pltpu.VMEM((PER // 128, 128), jnp.int32),
            pltpu.VMEM((ZCHUNK,), jnp.float32),
            pltpu.VMEM_SHARED((NP * NP,), jnp.float32),
        ],
    )
    def build(src_hbm, dst_hbm, w_hbm, out_hbm, src_v, dst_v, w_v, idx2d, z_v, acc):
        c = lax.axis_index("c")
        s = lax.axis_index("s")
        wid = c * 16 + s
        base = wid * PER
        pltpu.sync_copy(src_hbm.at[pl.ds(base, PER)], src_v)
        pltpu.sync_copy(dst_hbm.at[pl.ds(base, PER)], dst_v)
        pltpu.sync_copy(w_hbm.at[pl.ds(base, PER)], w_v)

        def zero_body(i, carry):
            z_v[pl.ds(i * 16, 16)] = jnp.zeros((16,), jnp.float32)
            return carry

        lax.fori_loop(0, ZCHUNK // 16, zero_body, 0)
        for k in range(TILE_WORDS // ZCHUNK):
            pltpu.sync_copy(z_v, acc.at[pl.ds(s * TILE_WORDS + k * ZCHUNK, ZCHUNK)])

        for j in range(PER // 16):
            v = dst_v[pl.ds(j * 16, 16)] * NP + src_v[pl.ds(j * 16, 16)]
            idx2d[j // 8, pl.ds((j % 8) * 16, 16)] = v

        plsc.subcore_barrier()
        for j in range(PER // 128):
            pltpu.sync_copy(
                w_v.at[pl.ds(j * 128, 128)], acc.at[idx2d.at[j]], add=True
            )
        plsc.subcore_barrier()

        for k in range(TILE_WORDS // ZCHUNK):
            pltpu.sync_copy(acc.at[pl.ds(s * TILE_WORDS + k * ZCHUNK, ZCHUNK)], z_v)
            pltpu.sync_copy(z_v, out_hbm.at[c, s, pl.ds(k * ZCHUNK, ZCHUNK)])

    return build(src_p, dst_p, w_p)


def _prep_body(a0_ref, a_ref):
    a = a0_ref[0] + a0_ref[1]
    acc = a[:N, :N]
    deg = jnp.sum(acc, axis=1) + 1.0
    dinv = lax.rsqrt(deg)
    scale = dinv[:, None] * dinv[None, :]
    r = lax.broadcasted_iota(jnp.int32, (N, N), 0)
    col = lax.broadcasted_iota(jnp.int32, (N, N), 1)
    eye = (r == col).astype(jnp.float32)
    a_ref[...] = (acc + eye) * scale


def _main_body(x_ref, w_ref, b_ref, gamma_ref, beta_ref, a_ref, o_ref):
    t = pl.program_id(0)
    wm = w_ref[...]
    xg = jnp.concatenate([x_ref[0, 0] @ wm, x_ref[1, 0] @ wm], axis=1)
    y = a_ref[...] @ xg
    bv = b_ref[...]
    y = y + jnp.concatenate([bv, bv])[None, :]
    n = 2.0 * N * bv.shape[0]
    mean = jnp.sum(y) / n
    d = y - mean
    var = jnp.sum(d * d) / n
    o = d * (lax.rsqrt(var + 1e-5) * gamma_ref[t]) + beta_ref[t]
    o = jnp.maximum(o, 0.0)
    o_ref[0, 0] = o[:, :32]
    o_ref[1, 0] = o[:, 32:]


def kernel(x, edge_index, W, b, gamma, beta):
    B, T, _, _ = x.shape
    src = edge_index[0]
    dst = edge_index[1]
    pad = EP - E
    src_p = jnp.concatenate([src, jnp.zeros((pad,), jnp.int32)])
    dst_p = jnp.concatenate([dst, jnp.zeros((pad,), jnp.int32)])
    w_p = jnp.concatenate([jnp.ones((E,), jnp.float32), jnp.zeros((pad,), jnp.float32)])

    a0 = _sc_build_a0(src_p, dst_p, w_p).reshape(2, NP, NP)

    a_mat = pl.pallas_call(
        _prep_body,
        out_shape=jax.ShapeDtypeStruct((N, N), jnp.float32),
    )(a0)

    c_out = W.shape[1]
    out = pl.pallas_call(
        _main_body,
        grid=(T,),
        in_specs=[
            pl.BlockSpec((B, 1, N, W.shape[0]), lambda t: (0, t, 0, 0)),
            pl.BlockSpec((W.shape[0], c_out), lambda t: (0, 0)),
            pl.BlockSpec((c_out,), lambda t: (0,)),
            pl.BlockSpec(memory_space=pltpu.SMEM),
            pl.BlockSpec(memory_space=pltpu.SMEM),
            pl.BlockSpec((N, N), lambda t: (0, 0)),
        ],
        out_specs=pl.BlockSpec((B, 1, N, c_out), lambda t: (0, t, 0, 0)),
        out_shape=jax.ShapeDtypeStruct((B, T, N, c_out), jnp.float32),
    )(x, W, b, gamma, beta, a_mat)
    return out


# speedup vs baseline: 12.3575x; 1.1703x over previous
import functools

import jax
import jax.numpy as jnp
from jax import lax
from jax.experimental import pallas as pl
from jax.experimental.pallas import tpu as pltpu
from jax.experimental.pallas import tpu_sc as plsc

N = 1000
NP = 1024
E = 16000
NW = 32
PER = 512
EP = NW * PER
TILE_WORDS = (NP * NP) // 16
ZCHUNK = 8192


def _sc_build_a0(src_p, dst_p, w_p):
    mesh = plsc.VectorSubcoreMesh(core_axis_name="c", subcore_axis_name="s")

    @functools.partial(
        pl.kernel,
        mesh=mesh,
        out_type=jax.ShapeDtypeStruct((2, 16, TILE_WORDS), jnp.float32),
        scratch_types=[
            pltpu.VMEM((PER,), jnp.int32),
            pltpu.VMEM((PER,), jnp.int32),
            pltpu.VMEM((PER,), jnp.float32),
            pltpu.VMEM((PER // 128, 128), jnp.int32),
            pltpu.VMEM((ZCHUNK,), jnp.float32),
            pltpu.VMEM_SHARED((NP * NP,), jnp.float32),
        ],
    )
    def build(src_hbm, dst_hbm, w_hbm, out_hbm, src_v, dst_v, w_v, idx2d, z_v, acc):
        c = lax.axis_index("c")
        s = lax.axis_index("s")
        wid = c * 16 + s
        base = wid * PER
        pltpu.sync_copy(src_hbm.at[pl.ds(base, PER)], src_v)
        pltpu.sync_copy(dst_hbm.at[pl.ds(base, PER)], dst_v)
        pltpu.sync_copy(w_hbm.at[pl.ds(base, PER)], w_v)

        def zero_body(i, carry):
            z_v[pl.ds(i * 16, 16)] = jnp.zeros((16,), jnp.float32)
            return carry

        lax.fori_loop(0, ZCHUNK // 16, zero_body, 0)
        for k in range(TILE_WORDS // ZCHUNK):
            pltpu.sync_copy(z_v, acc.at[pl.ds(s * TILE_WORDS + k * ZCHUNK, ZCHUNK)])

        for j in range(PER // 16):
            v = dst_v[pl.ds(j * 16, 16)] * NP + src_v[pl.ds(j * 16, 16)]
            idx2d[j // 8, pl.ds((j % 8) * 16, 16)] = v

        plsc.subcore_barrier()
        for j in range(PER // 128):
            pltpu.sync_copy(
                w_v.at[pl.ds(j * 128, 128)], acc.at[idx2d.at[j]], add=True
            )
        plsc.subcore_barrier()

        for k in range(TILE_WORDS // ZCHUNK):
            pltpu.sync_copy(acc.at[pl.ds(s * TILE_WORDS + k * ZCHUNK, ZCHUNK)], z_v)
            pltpu.sync_copy(z_v, out_hbm.at[c, s, pl.ds(k * ZCHUNK, ZCHUNK)])

    return build(src_p, dst_p, w_p)


def _prep_body(a0_ref, a_ref):
    a = a0_ref[0] + a0_ref[1]
    acc = a[:N, :N]
    deg = jnp.sum(acc, axis=1) + 1.0
    dinv = lax.rsqrt(deg)
    scale = dinv[:, None] * dinv[None, :]
    r = lax.broadcasted_iota(jnp.int32, (N, N), 0)
    col = lax.broadcasted_iota(jnp.int32, (N, N), 1)
    eye = (r == col).astype(jnp.float32)
    a_ref[...] = (acc + eye) * scale


TT = 4


def _main_body(x_ref, w_ref, b_ref, gamma_ref, beta_ref, a_ref, o_ref):
    tb = pl.program_id(0)
    wm = w_ref[...]
    xg = jnp.concatenate(
        [x_ref[bb, tt] @ wm for tt in range(TT) for bb in range(2)], axis=1
    )
    y = a_ref[...] @ xg
    bv = b_ref[...]
    c = bv.shape[0]
    y = y + jnp.concatenate([bv] * (2 * TT))[None, :]
    n = 2.0 * N * c
    for tt in range(TT):
        ys = y[:, tt * 2 * c:(tt + 1) * 2 * c]
        mean = jnp.sum(ys) / n
        d = ys - mean
        var = jnp.sum(d * d) / n
        t = tb * TT + tt
        o = d * (lax.rsqrt(var + 1e-5) * gamma_ref[t]) + beta_ref[t]
        o = jnp.maximum(o, 0.0)
        o_ref[0, tt] = o[:, :c]
        o_ref[1, tt] = o[:, c:]


def kernel(x, edge_index, W, b, gamma, beta):
    B, T, _, _ = x.shape
    src = edge_index[0]
    dst = edge_index[1]
    pad = EP - E
    src_p = jnp.concatenate([src, jnp.zeros((pad,), jnp.int32)])
    dst_p = jnp.concatenate([dst, jnp.zeros((pad,), jnp.int32)])
    w_p = jnp.concatenate([jnp.ones((E,), jnp.float32), jnp.zeros((pad,), jnp.float32)])

    a0 = _sc_build_a0(src_p, dst_p, w_p).reshape(2, NP, NP)

    a_mat = pl.pallas_call(
        _prep_body,
        out_shape=jax.ShapeDtypeStruct((N, N), jnp.float32),
    )(a0)

    c_out = W.shape[1]
    out = pl.pallas_call(
        _main_body,
        grid=(T // TT,),
        in_specs=[
            pl.BlockSpec((B, TT, N, W.shape[0]), lambda t: (0, t, 0, 0)),
            pl.BlockSpec((W.shape[0], c_out), lambda t: (0, 0)),
            pl.BlockSpec((c_out,), lambda t: (0,)),
            pl.BlockSpec(memory_space=pltpu.SMEM),
            pl.BlockSpec(memory_space=pltpu.SMEM),
            pl.BlockSpec((N, N), lambda t: (0, 0)),
        ],
        out_specs=pl.BlockSpec((B, TT, N, c_out), lambda t: (0, t, 0, 0)),
        out_shape=jax.ShapeDtypeStruct((B, T, N, c_out), jnp.float32),
    )(x, W, b, gamma, beta, a_mat)
    return out


# speedup vs baseline: 13.3637x; 1.0814x over previous
import functools

import jax
import jax.numpy as jnp
from jax import lax
from jax.experimental import pallas as pl
from jax.experimental.pallas import tpu as pltpu
from jax.experimental.pallas import tpu_sc as plsc

N = 1000
NP = 1024
E = 16000
NW = 32
PER = 512
EP = NW * PER
TILE_WORDS = (NP * NP) // 16
ZCHUNK = 8192


def _sc_build_a0(src_p, dst_p, w_p):
    mesh = plsc.VectorSubcoreMesh(core_axis_name="c", subcore_axis_name="s")

    @functools.partial(
        pl.kernel,
        mesh=mesh,
        out_type=jax.ShapeDtypeStruct((2, 16, TILE_WORDS), jnp.float32),
        scratch_types=[
            pltpu.VMEM((PER,), jnp.int32),
            pltpu.VMEM((PER,), jnp.int32),
            pltpu.VMEM((PER,), jnp.float32),
            pltpu.VMEM((PER // 128, 128), jnp.int32),
            pltpu.VMEM((ZCHUNK,), jnp.float32),
            pltpu.VMEM_SHARED((NP * NP,), jnp.float32),
        ],
    )
    def build(src_hbm, dst_hbm, w_hbm, out_hbm, src_v, dst_v, w_v, idx2d, z_v, acc):
        c = lax.axis_index("c")
        s = lax.axis_index("s")
        wid = c * 16 + s
        base = wid * PER
        pltpu.sync_copy(src_hbm.at[pl.ds(base, PER)], src_v)
        pltpu.sync_copy(dst_hbm.at[pl.ds(base, PER)], dst_v)
        pltpu.sync_copy(w_hbm.at[pl.ds(base, PER)], w_v)

        def zero_body(i, carry):
            z_v[pl.ds(i * 16, 16)] = jnp.zeros((16,), jnp.float32)
            return carry

        lax.fori_loop(0, ZCHUNK // 16, zero_body, 0)
        for k in range(TILE_WORDS // ZCHUNK):
            pltpu.sync_copy(z_v, acc.at[pl.ds(s * TILE_WORDS + k * ZCHUNK, ZCHUNK)])

        for j in range(PER // 16):
            v = dst_v[pl.ds(j * 16, 16)] * NP + src_v[pl.ds(j * 16, 16)]
            idx2d[j // 8, pl.ds((j % 8) * 16, 16)] = v

        plsc.subcore_barrier()
        for j in range(PER // 128):
            pltpu.sync_copy(
                w_v.at[pl.ds(j * 128, 128)], acc.at[idx2d.at[j]], add=True
            )
        plsc.subcore_barrier()

        for k in range(TILE_WORDS // ZCHUNK):
            pltpu.sync_copy(acc.at[pl.ds(s * TILE_WORDS + k * ZCHUNK, ZCHUNK)], z_v)
            pltpu.sync_copy(z_v, out_hbm.at[c, s, pl.ds(k * ZCHUNK, ZCHUNK)])

    return build(src_p, dst_p, w_p)


def _prep_body(a0_ref, a_ref):
    a = a0_ref[0] + a0_ref[1]
    acc = a[:N, :N]
    deg = jnp.sum(acc, axis=1) + 1.0
    dinv = lax.rsqrt(deg)
    scale = dinv[:, None] * dinv[None, :]
    r = lax.broadcasted_iota(jnp.int32, (N, N), 0)
    col = lax.broadcasted_iota(jnp.int32, (N, N), 1)
    eye = (r == col).astype(jnp.float32)
    a_ref[...] = ((acc + eye) * scale).astype(jnp.bfloat16)


TT = 4


def _main_body(x_ref, w_ref, b_ref, gamma_ref, beta_ref, a_ref, o_ref):
    tb = pl.program_id(0)
    wm = w_ref[...]
    xg = jnp.concatenate(
        [x_ref[bb, tt] @ wm for tt in range(TT) for bb in range(2)], axis=1
    )
    y = jnp.dot(
        a_ref[...], xg.astype(jnp.bfloat16), preferred_element_type=jnp.float32
    )
    bv = b_ref[...]
    c = bv.shape[0]
    y = y + jnp.concatenate([bv] * (2 * TT))[None, :]
    n = 2.0 * N * c
    for tt in range(TT):
        ys = y[:, tt * 2 * c:(tt + 1) * 2 * c]
        mean = jnp.sum(ys) / n
        d = ys - mean
        var = jnp.sum(d * d) / n
        t = tb * TT + tt
        o = d * (lax.rsqrt(var + 1e-5) * gamma_ref[t]) + beta_ref[t]
        o = jnp.maximum(o, 0.0)
        o_ref[0, tt] = o[:, :c]
        o_ref[1, tt] = o[:, c:]


def kernel(x, edge_index, W, b, gamma, beta):
    B, T, _, _ = x.shape
    src = edge_index[0]
    dst = edge_index[1]
    pad = EP - E
    src_p = jnp.concatenate([src, jnp.zeros((pad,), jnp.int32)])
    dst_p = jnp.concatenate([dst, jnp.zeros((pad,), jnp.int32)])
    w_p = jnp.concatenate([jnp.ones((E,), jnp.float32), jnp.zeros((pad,), jnp.float32)])

    a0 = _sc_build_a0(src_p, dst_p, w_p).reshape(2, NP, NP)

    a_mat = pl.pallas_call(
        _prep_body,
        out_shape=jax.ShapeDtypeStruct((N, N), jnp.bfloat16),
    )(a0)

    c_out = W.shape[1]
    out = pl.pallas_call(
        _main_body,
        grid=(T // TT,),
        in_specs=[
            pl.BlockSpec((B, TT, N, W.shape[0]), lambda t: (0, t, 0, 0)),
            pl.BlockSpec((W.shape[0], c_out), lambda t: (0, 0)),
            pl.BlockSpec((c_out,), lambda t: (0,)),
            pl.BlockSpec(memory_space=pltpu.SMEM),
            pl.BlockSpec(memory_space=pltpu.SMEM),
            pl.BlockSpec((N, N), lambda t: (0, 0)),
        ],
        out_specs=pl.BlockSpec((B, TT, N, c_out), lambda t: (0, t, 0, 0)),
        out_shape=jax.ShapeDtypeStruct((B, T, N, c_out), jnp.float32),
    )(x, W, b, gamma, beta, a_mat)
    return out
